# baseline (speedup 1.0000x reference)
import functools

import jax
import jax.numpy as jnp
from jax import lax
from jax.experimental import pallas as pl
from jax.experimental.pallas import tpu as pltpu
from jax.experimental.pallas import tpu_sc as plsc

VOCAB = 1000000
EMBED_DIM = 64
SEQ_LEN = 200
BATCH = 4096

NUM_CORES = 2
NUM_SUBCORES = 16
LANES = 16
NUM_WORKERS = NUM_CORES * NUM_SUBCORES

TOTAL = BATCH * SEQ_LEN
PER_WORKER = TOTAL // NUM_WORKERS
CHUNK = 512
SUB = 128
K = CHUNK // SUB
NUM_CHUNKS = PER_WORKER // CHUNK
PACK = 2 * EMBED_DIM
PROWS = CHUNK // 2
ROWS_PER_ITER = 2


def _sc_kernel(idx_hbm, table_hbm, pos_hbm, out_hbm, idx_v, rows_v, pack_v,
               pos_v, gsem):
  wid = lax.axis_index("s") * NUM_CORES + lax.axis_index("c")
  base = wid * PER_WORKER

  pltpu.sync_copy(pos_hbm, pos_v)

  def add_pack(off):
    s_row = off // BATCH
    pv = [pos_v[s_row, pl.ds(cb * LANES, LANES)]
          for cb in range(EMBED_DIM // LANES)]

    def body(r0, carry):
      for rr in range(ROWS_PER_ITER):
        prow = r0 * ROWS_PER_ITER + rr
        for c in range(PACK // LANES):
          src_row = 2 * prow + c // 4
          src_sl = pl.ds((c % 4) * LANES, LANES)
          pack_v[prow, pl.ds(c * LANES, LANES)] = (
              rows_v[src_row, src_sl] + pv[c % 4])
      return carry

    lax.fori_loop(0, PROWS // ROWS_PER_ITER, body, 0)

  for ci in range(NUM_CHUNKS):
    off = base + ci * CHUNK
    pltpu.sync_copy(idx_hbm.at[pl.ds(off, CHUNK)], idx_v)
    copies = []
    for j in range(K):
      copies.append(
          pltpu.async_copy(table_hbm.at[idx_v.at[pl.ds(j * SUB, SUB)]],
                           rows_v.at[pl.ds(j * SUB, SUB)], gsem))
    for c in copies:
      c.wait()
    add_pack(off)
    pltpu.sync_copy(pack_v, out_hbm.at[pl.ds(off // 2, PROWS)])


@jax.jit
def _run(idx_flat, word_table, pos_table):
  mesh = plsc.VectorSubcoreMesh(core_axis_name="c", subcore_axis_name="s")
  f = functools.partial(
      pl.kernel,
      mesh=mesh,
      compiler_params=pltpu.CompilerParams(use_tc_tiling_on_sc=False),
      out_type=jax.ShapeDtypeStruct((TOTAL // 2, PACK), jnp.float32),
      scratch_types=[
          pltpu.VMEM((CHUNK,), jnp.int32),
          pltpu.VMEM((CHUNK, EMBED_DIM), jnp.float32),
          pltpu.VMEM((PROWS, PACK), jnp.float32),
          pltpu.VMEM((SEQ_LEN, EMBED_DIM), jnp.float32),
          pltpu.SemaphoreType.DMA,
      ],
  )(_sc_kernel)
  return f(idx_flat, word_table, pos_table)


def kernel(input_idx, word_table, pos_table):
  idx_flat = input_idx.astype(jnp.int32).T.reshape(-1)
  out2 = _run(idx_flat, word_table, pos_table)
  o4 = out2.reshape(SEQ_LEN, BATCH // 2, 2, EMBED_DIM)
  return o4.transpose(1, 2, 0, 3).reshape(BATCH, SEQ_LEN, EMBED_DIM)


# speedup vs baseline: 1.1703x; 1.1703x over previous
import functools

import jax
import jax.numpy as jnp
from jax import lax
from jax.experimental import pallas as pl
from jax.experimental.pallas import tpu as pltpu
from jax.experimental.pallas import tpu_sc as plsc

VOCAB = 1000000
EMBED_DIM = 64
SEQ_LEN = 200
BATCH = 4096

NUM_CORES = 2
NUM_SUBCORES = 16
LANES = 16
NUM_WORKERS = NUM_CORES * NUM_SUBCORES

TOTAL = BATCH * SEQ_LEN
PER_WORKER = TOTAL // NUM_WORKERS
CHUNK = 1024
SUB = 128
K = CHUNK // SUB
NUM_CHUNKS = PER_WORKER // CHUNK
ROWS_PER_ITER = 4


def _sc_kernel(idx_hbm, table_hbm, pos_hbm, out_hbm, idx_v, rows_v, pos_v,
               gsem):
  wid = lax.axis_index("s") * NUM_CORES + lax.axis_index("c")
  base = wid * PER_WORKER

  pltpu.sync_copy(pos_hbm, pos_v)

  def add_pos(off):
    s_row = off // BATCH
    pv = [pos_v[s_row, pl.ds(cb * LANES, LANES)]
          for cb in range(EMBED_DIM // LANES)]

    def body(r0, carry):
      for rr in range(ROWS_PER_ITER):
        row = r0 * ROWS_PER_ITER + rr
        for cb in range(EMBED_DIM // LANES):
          sl = pl.ds(cb * LANES, LANES)
          rows_v[row, sl] = rows_v[row, sl] + pv[cb]
      return carry

    lax.fori_loop(0, CHUNK // ROWS_PER_ITER, body, 0)

  for ci in range(NUM_CHUNKS):
    off = base + ci * CHUNK
    pltpu.sync_copy(idx_hbm.at[pl.ds(off, CHUNK)], idx_v)
    copies = []
    for j in range(K):
      copies.append(
          pltpu.async_copy(table_hbm.at[idx_v.at[pl.ds(j * SUB, SUB)]],
                           rows_v.at[pl.ds(j * SUB, SUB)], gsem))
    for c in copies:
      c.wait()
    add_pos(off)
    pltpu.sync_copy(rows_v, out_hbm.at[pl.ds(off, CHUNK)])


@jax.jit
def _run(idx_flat, word_table, pos_table):
  mesh = plsc.VectorSubcoreMesh(core_axis_name="c", subcore_axis_name="s")
  f = functools.partial(
      pl.kernel,
      mesh=mesh,
      compiler_params=pltpu.CompilerParams(use_tc_tiling_on_sc=False),
      out_type=jax.ShapeDtypeStruct((TOTAL, EMBED_DIM), jnp.float32),
      scratch_types=[
          pltpu.VMEM((CHUNK,), jnp.int32),
          pltpu.VMEM((CHUNK, EMBED_DIM), jnp.float32),
          pltpu.VMEM((SEQ_LEN, EMBED_DIM), jnp.float32),
          pltpu.SemaphoreType.DMA,
      ],
  )(_sc_kernel)
  return f(idx_flat, word_table, pos_table)


def kernel(input_idx, word_table, pos_table):
  idx_flat = input_idx.astype(jnp.int32).T.reshape(-1)
  out = _run(idx_flat, word_table, pos_table)
  return out.reshape(SEQ_LEN, BATCH, EMBED_DIM).transpose(1, 0, 2)
